# baseline (speedup 1.0000x reference)
import jax
import jax.numpy as jnp
from jax import lax
from jax.experimental import pallas as pl
from jax.experimental.pallas import tpu as pltpu

B, C, N, FIN = 4, 5, 512, 128
H, D = 4, 64
HD = H * D
OUT = 16
P0, P1, P2 = B * C, 2 * B * C, 2 * B * C + B
LOG2E = 1.4426950408889634
_PREC = None
_NT = (((1,), (1,)), ((), ()))
_TN = (((0,), (0,)), ((), ()))


def _gat_core(h, adjm, nn, wpack, ppack, fin):
    iota_col = lax.broadcasted_iota(jnp.int32, (N, 1), 0)
    vcf = (iota_col < nn).astype(jnp.float32)
    feat = jnp.dot(h, wpack[0:fin], precision=_PREC)
    alt = wpack[fin:fin + H]
    art = wpack[fin + 8:fin + 8 + H]
    brow = wpack[fin + 16:fin + 17]
    el = lax.dot_general(feat, alt, _NT, precision=_PREC)
    ert = lax.dot_general(art, feat, _NT, precision=_PREC)
    cond = jnp.logical_and(adjm != 0.0, iota_col < nn)
    ones_col = jnp.ones((N, 1), dtype=jnp.float32)
    outs = []
    for hh in range(H):
        fh = feat[:, hh * D:(hh + 1) * D]
        s = el[:, hh:hh + 1] + ert[hh:hh + 1, :]
        s = jnp.maximum(s, 0.2 * s)
        p = lax.exp2(jnp.where(cond, s, -43.2808512))
        fe = jnp.concatenate([fh, ones_col], axis=1)
        oe = lax.dot_general(p, fe, _TN, precision=_PREC)
        outs.append(oe[:, :D] * (1.0 / oe[:, D:D + 1]))
    rst = jnp.concatenate(outs, axis=1) + brow
    z = jnp.where(rst > 0.0, rst, jnp.exp(rst) - 1.0)
    p1 = jnp.tanh(jnp.dot(z, ppack[0:HD], precision=_PREC)
                  + ppack[HD:HD + 1])
    pr = lax.dot_general(p1, ppack[HD + 8:HD + 9], _NT,
                         precision=_PREC)
    s_partial = jnp.sum(pr * vcf)
    return z, s_partial


def _beta_tile(s_tile, cnt):
    rows = lax.broadcasted_iota(jnp.int32, (8, 128), 0)
    t = jnp.where(rows < C, s_tile / cnt, -1e30)
    m = jnp.max(t, axis=0, keepdims=True)
    e = lax.exp2((t - m) * LOG2E)
    return e / jnp.sum(e, axis=0, keepdims=True)


def _combine(z_scr, b, beta_scr, sl):
    terms = [z_scr[b, cc] * beta_scr[sl, cc:cc + 1, 0:1] for cc in range(C)]
    return ((terms[0] + terms[1]) + (terms[2] + terms[3])) + terms[4]


def _cnt(nn_ref):
    return jnp.maximum(
        (nn_ref[0] + nn_ref[1] + nn_ref[2] + nn_ref[3]).astype(jnp.float32),
        1.0)


def _body(x_ref, adj_ref, nn_ref, w0_ref, p0_ref, w1_ref, p1_ref, wp_ref,
          out_ref, z0_scr, z1_scr, h_scr, s_scr, beta_scr):
    i = pl.program_id(0)
    rows = lax.broadcasted_iota(jnp.int32, (8, 128), 0)

    @pl.when(i == 0)
    def _init():
        s_scr[...] = jnp.zeros((2, 8, 128), dtype=jnp.float32)

    @pl.when(i < P0)
    def _layer0():
        b, c = i // C, i % C
        z, sp = _gat_core(x_ref[0, 0], adj_ref[0, 0], nn_ref[b],
                          w0_ref[0], p0_ref[...], FIN)
        z0_scr[b, c] = z
        s_scr[0] = s_scr[0] + jnp.where(rows == c, sp, 0.0)

    @pl.when(i == P0)
    def _beta0():
        beta_scr[0] = _beta_tile(s_scr[0], _cnt(nn_ref))

    @pl.when(jnp.logical_and(i >= P0, i < P1))
    def _layer1():
        j = i - P0
        b, c = j // C, j % C

        @pl.when(c == 0)
        def _mix():
            h_scr[...] = _combine(z0_scr, b, beta_scr, 0)

        z, sp = _gat_core(h_scr[...], adj_ref[0, 0], nn_ref[b],
                          w1_ref[0], p1_ref[...], HD)
        z1_scr[b, c] = z
        s_scr[1] = s_scr[1] + jnp.where(rows == c, sp, 0.0)

    @pl.when(i == P1)
    def _beta1():
        beta_scr[1] = _beta_tile(s_scr[1], _cnt(nn_ref))

    @pl.when(i >= P1)
    def _predict():
        b = i - P1
        acc = _combine(z1_scr, b, beta_scr, 1)
        res = (jnp.dot(acc, wp_ref[0:HD], precision=_PREC)
               + wp_ref[HD:HD + 1])
        vcf = (lax.broadcasted_iota(jnp.int32, (N, 1), 0)
               < nn_ref[b]).astype(jnp.float32)
        out_ref[0] = res * vcf


def _expand_attn_t(a):
    eye = jnp.eye(H, dtype=a.dtype)
    m = a[:, :, None, :] * eye[:, :, None]
    return m.reshape(C, H, HD)


def _pack_layer(W, al, ar, b):
    fin = W.shape[1]
    log2e = jnp.float32(LOG2E)
    alt = _expand_attn_t(al) * log2e
    art = _expand_attn_t(ar) * log2e
    pad4 = jnp.zeros((C, 4, HD), W.dtype)
    return jnp.concatenate(
        [W, alt, pad4, art, pad4, b.reshape(C, 1, HD)], axis=1)


def _pack_proj(Wp1, bp1, Wp2):
    pad7 = jnp.zeros((7, 128), Wp1.dtype)
    return jnp.concatenate(
        [Wp1, bp1.reshape(1, 128), pad7, Wp2.reshape(1, 128)], axis=0)


def kernel(x, adj, node_nums, W0, al0, ar0, b0, Wp1_0, bp1_0, Wp2_0,
           W1, al1, ar1, b1, Wp1_1, bp1_1, Wp2_1, Wpred, bpred):
    f32 = jnp.float32
    nn32 = node_nums.astype(jnp.int32)
    pack0 = _pack_layer(W0, al0, ar0, b0)
    pack1 = _pack_layer(W1, al1, ar1, b1)
    packp0 = _pack_proj(Wp1_0, bp1_0, Wp2_0)
    packp1 = _pack_proj(Wp1_1, bp1_1, Wp2_1)
    packpred = jnp.concatenate([Wpred, bpred.reshape(1, OUT)], axis=0)

    def full(shape):
        nd = len(shape)
        return pl.BlockSpec(shape, lambda i, _n=nd: (0,) * _n)

    logits = pl.pallas_call(
        _body,
        grid=(P2,),
        in_specs=[
            pl.BlockSpec((1, 1, N, FIN),
                         lambda i: (jnp.where(i < P0, i // C, 0), 0, 0, 0)),
            pl.BlockSpec((1, 1, N, N),
                         lambda i: (jnp.where(i < P1, (i % P0) // C, 0),
                                    jnp.where(i < P1, i % C, 0), 0, 0)),
            pl.BlockSpec(memory_space=pltpu.SMEM),
            pl.BlockSpec((1, FIN + 17, HD),
                         lambda i: (jnp.where(i < P0, i % C, 0), 0, 0)),
            full((HD + 9, 128)),
            pl.BlockSpec((1, HD + 17, HD),
                         lambda i: (jnp.where(
                             jnp.logical_and(i >= P0, i < P1), i % C, 0),
                             0, 0)),
            full((HD + 9, 128)),
            full((HD + 1, OUT)),
        ],
        out_specs=pl.BlockSpec(
            (1, N, OUT), lambda i: (jnp.where(i >= P1, i - P1, 0), 0, 0)),
        out_shape=jax.ShapeDtypeStruct((B, N, OUT), f32),
        scratch_shapes=[
            pltpu.VMEM((B, C, N, HD), f32),
            pltpu.VMEM((B, C, N, HD), f32),
            pltpu.VMEM((N, HD), f32),
            pltpu.VMEM((2, 8, 128), f32),
            pltpu.VMEM((2, 8, 128), f32),
        ],
        compiler_params=pltpu.CompilerParams(
            dimension_semantics=("arbitrary",)),
    )(x, adj, nn32, pack0, packp0, pack1, packp1, packpred)

    return logits


# speedup vs baseline: 1.0294x; 1.0294x over previous
import jax
import jax.numpy as jnp
from jax import lax
from jax.experimental import pallas as pl
from jax.experimental.pallas import tpu as pltpu

B, C, N, FIN = 4, 5, 512, 128
H, D = 4, 64
HD = H * D
OUT = 16
P0, P1, P2 = B * C, 2 * B * C, 2 * B * C + B
LOG2E = 1.4426950408889634
_PREC = None


def _gat_core(h, adjm, nn, W, Al, ArT, brow, Wp1, bp1, Wp2):
    iota_col = lax.broadcasted_iota(jnp.int32, (N, 1), 0)
    vcf = (iota_col < nn).astype(jnp.float32)
    feat = jnp.dot(h, W, precision=_PREC)
    el = jnp.dot(feat, Al, precision=_PREC)
    ert = lax.dot_general(ArT, feat, (((1,), (1,)), ((), ())),
                          precision=_PREC)
    cond = jnp.logical_and(adjm != 0.0, iota_col < nn)
    ones_col = jnp.ones((N, 1), dtype=jnp.float32)
    outs = []
    for hh in range(H):
        fh = feat[:, hh * D:(hh + 1) * D]
        s = el[:, hh:hh + 1] + ert[hh:hh + 1, :]
        s = jnp.maximum(s, 0.2 * s)
        p = lax.exp2(jnp.where(cond, s, -43.2808512))
        pb = p.astype(jnp.bfloat16)
        fe = jnp.concatenate([fh, ones_col], axis=1).astype(jnp.bfloat16)
        oe = lax.dot_general(pb, fe, (((0,), (0,)), ((), ())),
                             precision=_PREC,
                             preferred_element_type=jnp.float32)
        outs.append(oe[:, :D] * (1.0 / oe[:, D:D + 1]))
    rst = jnp.concatenate(outs, axis=1) + brow
    z = jnp.where(rst > 0.0, rst, jnp.exp(rst) - 1.0)
    p1 = jnp.tanh(jnp.dot(z, Wp1, precision=_PREC) + bp1)
    pr = jnp.dot(p1, Wp2, precision=_PREC)
    s_partial = jnp.sum(pr * vcf)
    return z, s_partial


def _beta_tile(s_tile, cnt):
    rows = lax.broadcasted_iota(jnp.int32, (8, 128), 0)
    t = jnp.where(rows < C, s_tile / cnt, -1e30)
    m = jnp.max(t, axis=0, keepdims=True)
    e = lax.exp2((t - m) * LOG2E)
    return e / jnp.sum(e, axis=0, keepdims=True)


def _body(x_ref, adj_ref, nn_ref,
          w0_ref, al0_ref, ar0_ref, b0_ref, wp10_ref, bp10_ref, wp20_ref,
          w1_ref, al1_ref, ar1_ref, b1_ref, wp11_ref, bp11_ref, wp21_ref,
          wpred_ref, bpred_ref, out_ref,
          z0_scr, z1_scr, h_scr, s_scr, beta_scr):
    i = pl.program_id(0)
    rows = lax.broadcasted_iota(jnp.int32, (8, 128), 0)
    cnt = jnp.maximum(
        (nn_ref[0] + nn_ref[1] + nn_ref[2] + nn_ref[3]).astype(jnp.float32),
        1.0)

    @pl.when(i == 0)
    def _init():
        s_scr[...] = jnp.zeros((2, 8, 128), dtype=jnp.float32)

    @pl.when(i < P0)
    def _layer0():
        b, c = i // C, i % C
        z, sp = _gat_core(x_ref[0, 0], adj_ref[0, 0], nn_ref[b],
                          w0_ref[0], al0_ref[0], ar0_ref[0], b0_ref[0],
                          wp10_ref[...], bp10_ref[...], wp20_ref[...])
        z0_scr[b, c] = z
        s_scr[0] = s_scr[0] + jnp.where(rows == c, sp, 0.0)

    @pl.when(i == P0)
    def _beta0():
        beta_scr[0] = _beta_tile(s_scr[0], cnt)

    @pl.when(jnp.logical_and(i >= P0, i < P1))
    def _layer1():
        j = i - P0
        b, c = j // C, j % C

        @pl.when(c == 0)
        def _combine():
            acc = z0_scr[b, 0] * beta_scr[0, 0:1, 0:1]
            for cc in range(1, C):
                acc = acc + z0_scr[b, cc] * beta_scr[0, cc:cc + 1, 0:1]
            h_scr[...] = acc

        z, sp = _gat_core(h_scr[...], adj_ref[0, 0], nn_ref[b],
                          w1_ref[0], al1_ref[0], ar1_ref[0], b1_ref[0],
                          wp11_ref[...], bp11_ref[...], wp21_ref[...])
        z1_scr[b, c] = z
        s_scr[1] = s_scr[1] + jnp.where(rows == c, sp, 0.0)

    @pl.when(i == P1)
    def _beta1():
        beta_scr[1] = _beta_tile(s_scr[1], cnt)

    @pl.when(i >= P1)
    def _predict():
        b = i - P1
        acc = z1_scr[b, 0] * beta_scr[1, 0:1, 0:1]
        for cc in range(1, C):
            acc = acc + z1_scr[b, cc] * beta_scr[1, cc:cc + 1, 0:1]
        res = jnp.dot(acc, wpred_ref[...], precision=_PREC) + bpred_ref[...]
        vcf = (lax.broadcasted_iota(jnp.int32, (N, 1), 0)
               < nn_ref[b]).astype(jnp.float32)
        out_ref[0] = res * vcf


def _expand_attn(a):
    eye = jnp.eye(H, dtype=a.dtype)
    m = a[:, :, :, None] * eye[None, :, None, :]
    return m.reshape(C, HD, H)


def kernel(x, adj, node_nums, W0, al0, ar0, b0, Wp1_0, bp1_0, Wp2_0,
           W1, al1, ar1, b1, Wp1_1, bp1_1, Wp2_1, Wpred, bpred):
    f32 = jnp.float32
    nn32 = node_nums.astype(jnp.int32)
    log2e = jnp.float32(LOG2E)
    al0m = _expand_attn(al0) * log2e
    ar0t = _expand_attn(ar0).transpose(0, 2, 1) * log2e
    al1m = _expand_attn(al1) * log2e
    ar1t = _expand_attn(ar1).transpose(0, 2, 1) * log2e
    b0r, b1r = b0.reshape(C, 1, HD), b1.reshape(C, 1, HD)
    bp1_0r, bp1_1r = bp1_0.reshape(1, 128), bp1_1.reshape(1, 128)
    bpredr = bpred.reshape(1, OUT)

    def jmap(i):
        return jnp.where(i >= P1, (i - P1) * C, jnp.where(i >= P0, i - P0, i))

    def full(shape):
        nd = len(shape)
        return pl.BlockSpec(shape, lambda i, _n=nd: (0,) * _n)

    def c0idx(i):
        return jnp.where(i < P0, i % C, 0)

    def c1idx(i):
        return jnp.where(jnp.logical_and(i >= P0, i < P1), (i - P0) % C, 0)

    logits = pl.pallas_call(
        _body,
        grid=(P2,),
        in_specs=[
            pl.BlockSpec((1, 1, N, FIN),
                         lambda i: (jnp.where(i < P0, i // C, 0), 0, 0, 0)),
            pl.BlockSpec((1, 1, N, N),
                         lambda i: (jnp.where(i < P1, jmap(i) // C, 0),
                                    jnp.where(i < P1, jmap(i) % C, 0), 0, 0)),
            pl.BlockSpec(memory_space=pltpu.SMEM),
            pl.BlockSpec((1, FIN, HD), lambda i: (c0idx(i), 0, 0)),
            pl.BlockSpec((1, HD, H), lambda i: (c0idx(i), 0, 0)),
            pl.BlockSpec((1, H, HD), lambda i: (c0idx(i), 0, 0)),
            pl.BlockSpec((1, 1, HD), lambda i: (c0idx(i), 0, 0)),
            full((HD, 128)), full((1, 128)), full((128, 1)),
            pl.BlockSpec((1, HD, HD), lambda i: (c1idx(i), 0, 0)),
            pl.BlockSpec((1, HD, H), lambda i: (c1idx(i), 0, 0)),
            pl.BlockSpec((1, H, HD), lambda i: (c1idx(i), 0, 0)),
            pl.BlockSpec((1, 1, HD), lambda i: (c1idx(i), 0, 0)),
            full((HD, 128)), full((1, 128)), full((128, 1)),
            full((HD, OUT)), full((1, OUT)),
        ],
        out_specs=pl.BlockSpec(
            (1, N, OUT), lambda i: (jnp.where(i >= P1, i - P1, 0), 0, 0)),
        out_shape=jax.ShapeDtypeStruct((B, N, OUT), f32),
        scratch_shapes=[
            pltpu.VMEM((B, C, N, HD), f32),
            pltpu.VMEM((B, C, N, HD), f32),
            pltpu.VMEM((N, HD), f32),
            pltpu.VMEM((2, 8, 128), f32),
            pltpu.VMEM((2, 8, 128), f32),
        ],
        compiler_params=pltpu.CompilerParams(
            dimension_semantics=("arbitrary",)),
    )(x, adj, nn32,
      W0, al0m, ar0t, b0r, Wp1_0, bp1_0r, Wp2_0,
      W1, al1m, ar1t, b1r, Wp1_1, bp1_1r, Wp2_1,
      Wpred, bpredr)

    return logits


# speedup vs baseline: 1.0820x; 1.0512x over previous
import jax
import jax.numpy as jnp
from jax import lax
from jax.experimental import pallas as pl
from jax.experimental.pallas import tpu as pltpu

B, C, N, FIN = 4, 5, 512, 128
H, D = 4, 64
HD = H * D
OUT = 16
LOG2E = 1.4426950408889634
_PREC = None


def _gat_core(h, adjm, nn, W, Al, ArT, brow, Wp1, bp1, Wp2):
    iota_col = lax.broadcasted_iota(jnp.int32, (N, 1), 0)
    vcf = (iota_col < nn).astype(jnp.float32)
    feat = jnp.dot(h, W, precision=_PREC)
    el = jnp.dot(feat, Al, precision=_PREC)
    ert = lax.dot_general(ArT, feat, (((1,), (1,)), ((), ())),
                          precision=_PREC)
    cond = jnp.logical_and(adjm != 0.0, iota_col < nn)
    ones_col = jnp.ones((N, 1), dtype=jnp.float32)
    outs = []
    for hh in range(H):
        fh = feat[:, hh * D:(hh + 1) * D]
        s = el[:, hh:hh + 1] + ert[hh:hh + 1, :]
        s = jnp.maximum(s, 0.2 * s)
        p = lax.exp2(jnp.where(cond, s, -43.2808512))
        fe = jnp.concatenate([fh, ones_col], axis=1)
        oe = lax.dot_general(p, fe, (((0,), (0,)), ((), ())),
                             precision=_PREC)
        outs.append(oe[:, :D] * (1.0 / oe[:, D:D + 1]))
    rst = jnp.concatenate(outs, axis=1) + brow
    z = jnp.where(rst > 0.0, rst, jnp.exp(rst) - 1.0)
    p1 = jnp.tanh(jnp.dot(z, Wp1, precision=_PREC) + bp1)
    pr = jnp.dot(p1, Wp2, precision=_PREC)
    s_partial = jnp.sum(pr * vcf)
    return z, s_partial


def _beta_tile(s_tile, cnt):
    rows = lax.broadcasted_iota(jnp.int32, (8, 128), 0)
    t = jnp.where(rows < C, s_tile / cnt, -1e30)
    m = jnp.max(t, axis=0, keepdims=True)
    e = lax.exp2((t - m) * LOG2E)
    return e / jnp.sum(e, axis=0, keepdims=True)


def _body(x_ref, adj_ref, nn_ref,
          w0_ref, al0_ref, ar0_ref, b0_ref, wp10_ref, bp10_ref, wp20_ref,
          w1_ref, al1_ref, ar1_ref, b1_ref, wp11_ref, bp11_ref, wp21_ref,
          wpred_ref, bpred_ref, out_ref,
          z_scr, s_scr, beta_scr):
    i = pl.program_id(0)
    rows = lax.broadcasted_iota(jnp.int32, (8, 128), 0)
    cnt = jnp.maximum(
        (nn_ref[0] + nn_ref[1] + nn_ref[2] + nn_ref[3]).astype(jnp.float32),
        1.0)

    @pl.when(i == 0)
    def _init():
        s_scr[...] = jnp.zeros((2, 8, 128), dtype=jnp.float32)

    @pl.when(i < B)
    def _layer0():
        b = i
        acc = jnp.zeros((8, 128), dtype=jnp.float32)
        for c in range(C):
            z, sp = _gat_core(x_ref[0, 0], adj_ref[0, c], nn_ref[b],
                              w0_ref[c], al0_ref[c], ar0_ref[c],
                              b0_ref[c], wp10_ref[...], bp10_ref[...],
                              wp20_ref[...])
            z_scr[0, b, c] = z
            acc = acc + jnp.where(rows == c, sp, 0.0)
        s_scr[0] = s_scr[0] + acc

    @pl.when(i == B)
    def _beta0():
        beta_scr[0] = _beta_tile(s_scr[0], cnt)

    @pl.when(jnp.logical_and(i >= B, i < 2 * B))
    def _layer1():
        b = i - B
        hmix = z_scr[0, b, 0] * beta_scr[0, 0:1, 0:1]
        for cc in range(1, C):
            hmix = hmix + z_scr[0, b, cc] * beta_scr[0, cc:cc + 1, 0:1]
        acc = jnp.zeros((8, 128), dtype=jnp.float32)
        for c in range(C):
            z, sp = _gat_core(hmix, adj_ref[0, c], nn_ref[b],
                              w1_ref[c], al1_ref[c], ar1_ref[c],
                              b1_ref[c], wp11_ref[...], bp11_ref[...],
                              wp21_ref[...])
            z_scr[1, b, c] = z
            acc = acc + jnp.where(rows == c, sp, 0.0)
        s_scr[1] = s_scr[1] + acc

    @pl.when(i == 2 * B)
    def _beta1():
        beta_scr[1] = _beta_tile(s_scr[1], cnt)

    @pl.when(i >= 2 * B)
    def _predict():
        b = i - 2 * B
        acc = z_scr[1, b, 0] * beta_scr[1, 0:1, 0:1]
        for cc in range(1, C):
            acc = acc + z_scr[1, b, cc] * beta_scr[1, cc:cc + 1, 0:1]
        res = jnp.dot(acc, wpred_ref[...], precision=_PREC) + bpred_ref[...]
        vcf = (lax.broadcasted_iota(jnp.int32, (N, 1), 0)
               < nn_ref[b]).astype(jnp.float32)
        out_ref[0] = res * vcf


def _expand_attn(a):
    eye = jnp.eye(H, dtype=a.dtype)
    m = a[:, :, :, None] * eye[None, :, None, :]
    return m.reshape(C, HD, H)


def kernel(x, adj, node_nums, W0, al0, ar0, b0, Wp1_0, bp1_0, Wp2_0,
           W1, al1, ar1, b1, Wp1_1, bp1_1, Wp2_1, Wpred, bpred):
    f32 = jnp.float32
    nn32 = node_nums.astype(jnp.int32)
    log2e = jnp.float32(LOG2E)
    al0m = _expand_attn(al0) * log2e
    ar0t = _expand_attn(ar0).transpose(0, 2, 1) * log2e
    al1m = _expand_attn(al1) * log2e
    ar1t = _expand_attn(ar1).transpose(0, 2, 1) * log2e
    b0r, b1r = b0.reshape(C, 1, HD), b1.reshape(C, 1, HD)
    bp1_0r, bp1_1r = bp1_0.reshape(1, 128), bp1_1.reshape(1, 128)
    bpredr = bpred.reshape(1, OUT)

    def full(shape):
        nd = len(shape)
        return pl.BlockSpec(shape, lambda i, _n=nd: (0,) * _n)

    logits = pl.pallas_call(
        _body,
        grid=(3 * B,),
        in_specs=[
            pl.BlockSpec((1, 1, N, FIN),
                         lambda i: (jnp.where(i < B, i, 0), 0, 0, 0)),
            pl.BlockSpec((1, C, N, N),
                         lambda i: (jnp.where(i < 2 * B, i % B, 0), 0, 0, 0)),
            pl.BlockSpec(memory_space=pltpu.SMEM),
            full((C, FIN, HD)), full((C, HD, H)), full((C, H, HD)),
            full((C, 1, HD)),
            full((HD, 128)), full((1, 128)), full((128, 1)),
            full((C, HD, HD)), full((C, HD, H)), full((C, H, HD)),
            full((C, 1, HD)),
            full((HD, 128)), full((1, 128)), full((128, 1)),
            full((HD, OUT)), full((1, OUT)),
        ],
        out_specs=pl.BlockSpec(
            (1, N, OUT), lambda i: (jnp.where(i >= 2 * B, i - 2 * B, 0), 0, 0)),
        out_shape=jax.ShapeDtypeStruct((B, N, OUT), f32),
        scratch_shapes=[
            pltpu.VMEM((2, B, C, N, HD), f32),
            pltpu.VMEM((2, 8, 128), f32),
            pltpu.VMEM((2, 8, 128), f32),
        ],
        compiler_params=pltpu.CompilerParams(
            dimension_semantics=("arbitrary",)),
    )(x, adj, nn32,
      W0, al0m, ar0t, b0r, Wp1_0, bp1_0r, Wp2_0,
      W1, al1m, ar1t, b1r, Wp1_1, bp1_1r, Wp2_1,
      Wpred, bpredr)

    return logits


# speedup vs baseline: 1.0971x; 1.0140x over previous
import jax
import jax.numpy as jnp
from jax import lax
from jax.experimental import pallas as pl
from jax.experimental.pallas import tpu as pltpu

B, C, N, FIN = 4, 5, 512, 128
H, D = 4, 64
HD = H * D
OUT = 16
LOG2E = 1.4426950408889634
_PREC = None


def _gat_core(h, adjm, nn, W, Al, ArT, brow, Wp1, bp1, Wp2):
    iota_col = lax.broadcasted_iota(jnp.int32, (N, 1), 0)
    vcf = (iota_col < nn).astype(jnp.float32)
    feat = jnp.dot(h, W, precision=_PREC)
    el = jnp.dot(feat, Al, precision=_PREC)
    ert = lax.dot_general(ArT, feat, (((1,), (1,)), ((), ())),
                          precision=_PREC)
    amask = jnp.where(jnp.logical_and(adjm != 0.0, iota_col < nn),
                      0.0, -43.2808512)
    ones_col = jnp.ones((N, 1), dtype=jnp.float32)
    outs = []
    for hh in range(H):
        fh = feat[:, hh * D:(hh + 1) * D]
        s = el[:, hh:hh + 1] + ert[hh:hh + 1, :]
        s = jnp.maximum(s, 0.2 * s)
        p = lax.exp2(s + amask)
        fe = jnp.concatenate([fh, ones_col], axis=1)
        oe = lax.dot_general(p, fe, (((0,), (0,)), ((), ())),
                             precision=_PREC)
        outs.append(oe[:, :D] * (1.0 / oe[:, D:D + 1]))
    rst = jnp.concatenate(outs, axis=1) + brow
    z = jnp.where(rst > 0.0, rst, jnp.exp(rst) - 1.0)
    p1 = jnp.tanh(jnp.dot(z, Wp1, precision=_PREC) + bp1)
    pr = jnp.dot(p1, Wp2, precision=_PREC)
    s_partial = jnp.sum(pr * vcf)
    return z, s_partial


def _beta_tile(s_tile, cnt):
    rows = lax.broadcasted_iota(jnp.int32, (8, 128), 0)
    t = jnp.where(rows < C, s_tile / cnt, -1e30)
    m = jnp.max(t, axis=0, keepdims=True)
    e = lax.exp2((t - m) * LOG2E)
    return e / jnp.sum(e, axis=0, keepdims=True)


def _body(x_ref, adj_ref, nn_ref,
          w0_ref, al0_ref, ar0_ref, b0_ref, wp10_ref, bp10_ref, wp20_ref,
          w1_ref, al1_ref, ar1_ref, b1_ref, wp11_ref, bp11_ref, wp21_ref,
          wpred_ref, bpred_ref, out_ref,
          z_scr, s_scr, beta_scr):
    i = pl.program_id(0)
    rows = lax.broadcasted_iota(jnp.int32, (8, 128), 0)
    cnt = jnp.maximum(
        (nn_ref[0] + nn_ref[1] + nn_ref[2] + nn_ref[3]).astype(jnp.float32),
        1.0)

    @pl.when(i == 0)
    def _init():
        s_scr[...] = jnp.zeros((2, 8, 128), dtype=jnp.float32)

    @pl.when(i < B)
    def _layer0():
        b = i
        acc = jnp.zeros((8, 128), dtype=jnp.float32)
        for c in range(C):
            z, sp = _gat_core(x_ref[0, 0], adj_ref[0, c], nn_ref[b],
                              w0_ref[c], al0_ref[c], ar0_ref[c],
                              b0_ref[c], wp10_ref[...], bp10_ref[...],
                              wp20_ref[...])
            z_scr[0, b, c] = z
            acc = acc + jnp.where(rows == c, sp, 0.0)
        s_scr[0] = s_scr[0] + acc

    @pl.when(i == B)
    def _beta0():
        beta_scr[0] = _beta_tile(s_scr[0], cnt)

    @pl.when(jnp.logical_and(i >= B, i < 2 * B))
    def _layer1():
        b = i - B
        hmix = z_scr[0, b, 0] * beta_scr[0, 0:1, 0:1]
        for cc in range(1, C):
            hmix = hmix + z_scr[0, b, cc] * beta_scr[0, cc:cc + 1, 0:1]
        acc = jnp.zeros((8, 128), dtype=jnp.float32)
        for c in range(C):
            z, sp = _gat_core(hmix, adj_ref[0, c], nn_ref[b],
                              w1_ref[c], al1_ref[c], ar1_ref[c],
                              b1_ref[c], wp11_ref[...], bp11_ref[...],
                              wp21_ref[...])
            z_scr[1, b, c] = z
            acc = acc + jnp.where(rows == c, sp, 0.0)
        s_scr[1] = s_scr[1] + acc

    @pl.when(i == 2 * B)
    def _beta1():
        beta_scr[1] = _beta_tile(s_scr[1], cnt)

    @pl.when(i >= 2 * B)
    def _predict():
        b = i - 2 * B
        acc = z_scr[1, b, 0] * beta_scr[1, 0:1, 0:1]
        for cc in range(1, C):
            acc = acc + z_scr[1, b, cc] * beta_scr[1, cc:cc + 1, 0:1]
        res = jnp.dot(acc, wpred_ref[...], precision=_PREC) + bpred_ref[...]
        vcf = (lax.broadcasted_iota(jnp.int32, (N, 1), 0)
               < nn_ref[b]).astype(jnp.float32)
        out_ref[0] = res * vcf


def _expand_attn(a):
    eye = jnp.eye(H, dtype=a.dtype)
    m = a[:, :, :, None] * eye[None, :, None, :]
    return m.reshape(C, HD, H)


def kernel(x, adj, node_nums, W0, al0, ar0, b0, Wp1_0, bp1_0, Wp2_0,
           W1, al1, ar1, b1, Wp1_1, bp1_1, Wp2_1, Wpred, bpred):
    f32 = jnp.float32
    nn32 = node_nums.astype(jnp.int32)
    log2e = jnp.float32(LOG2E)
    al0m = _expand_attn(al0) * log2e
    ar0t = _expand_attn(ar0).transpose(0, 2, 1) * log2e
    al1m = _expand_attn(al1) * log2e
    ar1t = _expand_attn(ar1).transpose(0, 2, 1) * log2e
    b0r, b1r = b0.reshape(C, 1, HD), b1.reshape(C, 1, HD)
    bp1_0r, bp1_1r = bp1_0.reshape(1, 128), bp1_1.reshape(1, 128)
    bpredr = bpred.reshape(1, OUT)

    def full(shape):
        nd = len(shape)
        return pl.BlockSpec(shape, lambda i, _n=nd: (0,) * _n)

    logits = pl.pallas_call(
        _body,
        grid=(3 * B,),
        in_specs=[
            pl.BlockSpec((1, 1, N, FIN),
                         lambda i: (jnp.where(i < B, i, 0), 0, 0, 0)),
            pl.BlockSpec((1, C, N, N),
                         lambda i: (jnp.where(i < 2 * B, i % B, 0), 0, 0, 0)),
            pl.BlockSpec(memory_space=pltpu.SMEM),
            full((C, FIN, HD)), full((C, HD, H)), full((C, H, HD)),
            full((C, 1, HD)),
            full((HD, 128)), full((1, 128)), full((128, 1)),
            full((C, HD, HD)), full((C, HD, H)), full((C, H, HD)),
            full((C, 1, HD)),
            full((HD, 128)), full((1, 128)), full((128, 1)),
            full((HD, OUT)), full((1, OUT)),
        ],
        out_specs=pl.BlockSpec(
            (1, N, OUT), lambda i: (jnp.where(i >= 2 * B, i - 2 * B, 0), 0, 0)),
        out_shape=jax.ShapeDtypeStruct((B, N, OUT), f32),
        scratch_shapes=[
            pltpu.VMEM((2, B, C, N, HD), f32),
            pltpu.VMEM((2, 8, 128), f32),
            pltpu.VMEM((2, 8, 128), f32),
        ],
        compiler_params=pltpu.CompilerParams(
            dimension_semantics=("arbitrary",)),
    )(x, adj, nn32,
      W0, al0m, ar0t, b0r, Wp1_0, bp1_0r, Wp2_0,
      W1, al1m, ar1t, b1r, Wp1_1, bp1_1r, Wp2_1,
      Wpred, bpredr)

    return logits
